# Initial kernel scaffold; baseline (speedup 1.0000x reference)
import jax
import jax.numpy as jnp
from jax.experimental import pallas as pl


def kernel(inputs, targets, entire_inputs, edge_index, M1, M2, W1, W2):
    raise NotImplementedError("write your pallas kernel here")


# speedup vs baseline: 4.6270x; 4.6270x over previous
import jax
import jax.numpy as jnp
from jax.experimental import pallas as pl
from jax.experimental.pallas import tpu as pltpu

N = 2048
D_EMB = 256
K = 30
ALPHA = 3.0
H = 64
ROWS = 256


def _fused_body(m1_ref, m2_ref, m1t_ref, m2t_ref, x_ref, w1_ref, w2_ref,
                adj_ref, out_ref):
    m1 = m1_ref[...]
    m2 = m2_ref[...]
    logits = jnp.dot(m1, m2t_ref[...], preferred_element_type=jnp.float32)
    logits = logits - jnp.dot(m2, m1t_ref[...],
                              preferred_element_type=jnp.float32)
    theta = jnp.maximum(jnp.tanh(ALPHA * logits), 0.0)

    neg = jnp.float32(-3e38)
    t = jnp.max(logits, axis=1, keepdims=True)
    for _ in range(K - 1):
        c = jnp.sum(jnp.where(logits >= t, 1.0, 0.0), axis=1, keepdims=True)
        nxt = jnp.max(jnp.where(logits < t, logits, neg), axis=1,
                      keepdims=True)
        t = jnp.where(c < K, nxt, t)

    adj = jnp.where(logits >= t, theta, 0.0)
    adj_ref[...] = adj

    deg = jnp.sum(adj, axis=1, keepdims=True) + 1e-8
    an = adj / deg
    prop = jnp.dot(an, x_ref[...], preferred_element_type=jnp.float32)

    bt = prop.shape[1] // 2
    p0 = prop[:, :bt]
    p1 = prop[:, bt:]
    acc0 = jnp.zeros_like(p0)
    acc1 = jnp.zeros_like(p1)
    for h in range(H):
        hh = jnp.tanh(p0 * w1_ref[0:1, h:h + 1] + p1 * w1_ref[1:2, h:h + 1])
        acc0 = acc0 + hh * w2_ref[h:h + 1, 0:1]
        acc1 = acc1 + hh * w2_ref[h:h + 1, 1:2]
    out_ref[:, :bt] = acc0
    out_ref[:, bt:] = acc1


def kernel(inputs, targets, entire_inputs, edge_index, M1, M2, W1, W2):
    B, T, n, F = inputs.shape
    BT = B * T
    X = jnp.transpose(inputs, (2, 3, 0, 1)).reshape(n, F * BT)
    M1T = M1.T
    M2T = M2.T

    grid = (N // ROWS,)
    adj, out = pl.pallas_call(
        _fused_body,
        grid=grid,
        in_specs=[
            pl.BlockSpec((ROWS, D_EMB), lambda i: (i, 0)),
            pl.BlockSpec((ROWS, D_EMB), lambda i: (i, 0)),
            pl.BlockSpec((D_EMB, N), lambda i: (0, 0)),
            pl.BlockSpec((D_EMB, N), lambda i: (0, 0)),
            pl.BlockSpec((N, F * BT), lambda i: (0, 0)),
            pl.BlockSpec((2, H), lambda i: (0, 0)),
            pl.BlockSpec((H, 2), lambda i: (0, 0)),
        ],
        out_specs=[
            pl.BlockSpec((ROWS, N), lambda i: (i, 0)),
            pl.BlockSpec((ROWS, F * BT), lambda i: (i, 0)),
        ],
        out_shape=[
            jax.ShapeDtypeStruct((N, N), jnp.float32),
            jax.ShapeDtypeStruct((N, F * BT), jnp.float32),
        ],
    )(M1, M2, M1T, M2T, X, W1, W2)

    outputs = out.reshape(n, F, B, T).transpose(2, 3, 0, 1)
    return (adj, outputs)
